# baseline (speedup 1.0000x reference)
import jax
import jax.numpy as jnp
from jax import lax
from jax.experimental import pallas as pl
from jax.experimental.pallas import tpu as pltpu
from jax.experimental.pallas import tpu_sc as plsc

_N = 512
_F0 = 8
_T = 8
_B = 4
_E = 8192
_C = 64
_K = 3
_f32 = jnp.float32
_HI = lax.Precision.HIGHEST
_DEF = lax.Precision.DEFAULT

_NBP = 18


def _mm(a, b, prec=_DEF):
    return lax.dot_general(a, b, (((a.ndim - 1,), (0,)), ((), ())),
                           precision=prec, preferred_element_type=_f32)


def _mm00(a, b, prec=_DEF):
    return lax.dot_general(a, b, (((0,), (0,)), ((), ())),
                           precision=prec, preferred_element_type=_f32)


def _mm11(a, b, prec=_DEF):
    return lax.dot_general(a, b, (((1,), (1,)), ((), ())),
                           precision=prec, preferred_element_type=_f32)


def _iota_r(n):
    return lax.broadcasted_iota(jnp.int32, (n, 1), 0).astype(_f32)


def _iota_c(n):
    return lax.broadcasted_iota(jnp.int32, (1, n), 1).astype(_f32)


def _pool_body(x_ref, w_ref, out_ref, rank_ref):
    xb = x_ref[0]
    w = w_ref[...]
    nrm = jnp.sqrt(jnp.sum(w * w))
    s = jnp.tanh(jnp.sum(xb * w, axis=1, keepdims=True) / nrm)
    eye = (_iota_r(_N) == _iota_c(_N)).astype(_f32)
    s_row = _mm00(s, eye)
    gt = (s > s_row).astype(_f32)
    tie = ((s == s_row) & (_iota_r(_N) < _iota_c(_N))).astype(_f32)
    rank = jnp.sum(gt + tie, axis=0, keepdims=True)
    perm = (rank == _iota_r(_N)).astype(_f32)
    pooled = _mm(perm, xb * s)
    out_ref[0] = jnp.concatenate(
        [pooled, jnp.zeros((_N, _C - _F0), _f32)], axis=1)
    rank_ref[0] = rank


def _pool_call(x_tn, w):
    return pl.pallas_call(
        _pool_body,
        grid=(_B * _T,),
        in_specs=[
            pl.BlockSpec((1, _N, _F0), lambda i: (i, 0, 0)),
            pl.BlockSpec((1, _F0), lambda i: (0, 0)),
        ],
        out_specs=[
            pl.BlockSpec((1, _N, _C), lambda i: (i, 0, 0)),
            pl.BlockSpec((1, 1, _N), lambda i: (i, 0, 0)),
        ],
        out_shape=[
            jax.ShapeDtypeStruct((_B * _T, _N, _C), _f32),
            jax.ShapeDtypeStruct((_B * _T, 1, _N), _f32),
        ],
    )(x_tn, w)


_NN = _N * _N
_EPW = _E // 32
_SLC = _NN // 16


def _sc_cnt_body(rows_hbm, cols_hbm, zeros_hbm, out_hbm,
                 rows_v, cols_v, ones_v, cnt_sh):
    c = lax.axis_index("c")
    s = lax.axis_index("s")
    wid = c * 16 + s
    pltpu.sync_copy(zeros_hbm.at[pl.ds(s * _SLC, _SLC)],
                    cnt_sh.at[pl.ds(s * _SLC, _SLC)])
    plsc.subcore_barrier()
    pltpu.sync_copy(rows_hbm.at[pl.ds(wid * _EPW, _EPW)], rows_v)
    pltpu.sync_copy(cols_hbm.at[pl.ds(wid * _EPW, _EPW)], cols_v)
    ones_v[...] = jnp.ones((16,), _f32)

    def g_body(g, carry):
        rvec = rows_v[pl.ds(g * 16, 16)]
        cvec = cols_v[pl.ds(g * 16, 16)]
        idx = rvec * _N + cvec
        pltpu.sync_copy(ones_v, cnt_sh.at[idx], add=True)
        return carry
    lax.fori_loop(0, _EPW // 16, g_body, 0)
    plsc.subcore_barrier()
    pltpu.sync_copy(cnt_sh.at[pl.ds(s * _SLC, _SLC)],
                    out_hbm.at[c, pl.ds(s * _SLC, _SLC)])


def _sc_cnt_call(rows_i, cols_i):
    mesh = plsc.VectorSubcoreMesh(core_axis_name="c", subcore_axis_name="s")
    return pl.kernel(
        _sc_cnt_body,
        out_type=jax.ShapeDtypeStruct((2, _NN), _f32),
        mesh=mesh,
        scratch_types=[
            pltpu.VMEM((_EPW,), jnp.int32),
            pltpu.VMEM((_EPW,), jnp.int32),
            pltpu.VMEM((16,), _f32),
            pltpu.VMEM_SHARED((_NN,), _f32),
        ],
    )(rows_i, cols_i, jnp.zeros((_NN,), _f32))


def _adj_body(c2_ref, rk_ref, l_ref):
    eye = (_iota_r(_N) == _iota_c(_N)).astype(_f32)
    cnt = c2_ref[0] + c2_ref[1]
    deg = jnp.sum(cnt, axis=1, keepdims=True)
    dis = jnp.where(deg > 0, 1.0 / jnp.sqrt(deg), 0.0)
    dis_row = _mm00(dis, eye)
    l_ref[4] = -(dis * dis_row) * cnt
    for t in range(4):
        rk = rk_ref[t:t + 1, :]
        perm = (rk == _iota_r(_N)).astype(_f32)
        tmp = _mm(perm, cnt, _DEF)
        ct = _mm11(tmp, perm, _DEF)
        dt_col = _mm(perm, dis)
        dt_row = _mm00(dt_col, eye)
        l_ref[t] = -(dt_col * dt_row) * ct


def _adj_call(cnt2, ranks7):
    return pl.pallas_call(
        _adj_body,
        grid=(1,),
        in_specs=[
            pl.BlockSpec((2, _N, _N), lambda i: (0, 0, 0)),
            pl.BlockSpec((_B, _N), lambda i: (0, 0)),
        ],
        out_specs=pl.BlockSpec((5, _N, _N), lambda i: (0, 0, 0)),
        out_shape=jax.ShapeDtypeStruct((5, _N, _N), _f32),
    )(cnt2, ranks7)


def _blocks_body(*refs):
    a0_ref, l_ref, imu_ref = refs[0:3]
    bps = refs[3:3 + 3 * _NBP]
    mlp = refs[3 + 3 * _NBP:3 + 3 * _NBP + 6]
    o_ref = refs[3 + 3 * _NBP + 6]
    a_ref, h_ref = refs[3 + 3 * _NBP + 7:]

    eye = (_iota_r(_N) == _iota_c(_N)).astype(_f32)

    for blk in range(3):
        (u1_ref, u2_ref, u3_ref, be_ref, ve_ref, w1_ref, w2_ref, w3_ref,
         bs_ref, vs_ref, th_ref, cb_ref, wt_ref, bt_ref, wr_ref, br_ref,
         g_ref, be2_ref) = bps[_NBP * blk:_NBP * (blk + 1)]
        src = a0_ref if blk == 0 else a_ref
        cb = cb_ref[...]
        btv = bt_ref[...]
        brv = br_ref[...]
        g = g_ref[...]
        be2 = be2_ref[...]
        u1 = u1_ref[...]
        u3 = u3_ref[...]
        w3 = w3_ref[...]

        def b_body(b, _):
            Ab = src[b]
            lhs1 = jnp.sum(Ab * u1[:, :, None], axis=1)
            lhs2 = _mm(lhs1, u2_ref[...])
            rhs = jnp.sum(Ab * u3[None, :, :], axis=2)
            e1 = _mm11(lhs2, rhs)
            esig = jax.nn.sigmoid(e1 + be_ref[...])
            eatt = _mm(ve_ref[...], esig)
            mx = jnp.max(eatt, axis=0, keepdims=True)
            ex = jnp.exp(eatt - mx)
            eatt = ex / jnp.sum(ex, axis=0, keepdims=True)
            c = _mm(eatt, w1_ref[...])
            l2a = jnp.sum(Ab * c[:, :, None], axis=0)
            l2b = _mm(l2a, w2_ref[...])
            r0 = jnp.sum(Ab * w3[None, :, :], axis=2)
            r2 = _mm00(eatt, r0)
            s1 = jax.nn.sigmoid(_mm(l2b, r2) + bs_ref[...])
            s2 = _mm(vs_ref[...], s1)
            mx2 = jnp.max(s2, axis=0, keepdims=True)
            ex2 = jnp.exp(s2 - mx2)
            S = ex2 / jnp.sum(ex2, axis=0, keepdims=True)
            diag = jnp.sum(S * eye, axis=1, keepdims=True)

            def t_cheb(t, _c):
                lt = l_ref[jnp.minimum(t, 4)]
                xt = src[b, t]
                t0 = xt * diag
                out = _mm(t0, th_ref[0]) + cb
                t1 = _mm00(lt * S, t0)
                out = out + _mm(t1, th_ref[1])
                t2 = 2.0 * _mm00(lt, t1) - t0
                out = out + _mm(t2, th_ref[2])
                h_ref[t] = jnp.maximum(out, 0.0)
                return 0
            lax.fori_loop(0, _T, t_cheb, 0)

            def t_conv(t, _c):
                acc = _mm11(src[b, t], wr_ref[...])
                for dt in range(3):
                    stp = t + dt - 1
                    valid = jnp.logical_and(stp >= 0, stp < _T)
                    hs = h_ref[jnp.clip(stp, 0, _T - 1)]
                    acc = acc + jnp.where(valid, 1.0, 0.0) * _mm11(hs, wt_ref[:, :, 0, dt])
                z = jnp.maximum(acc + btv + brv, 0.0)
                mu = jnp.mean(z, axis=1, keepdims=True)
                var = jnp.mean((z - mu) * (z - mu), axis=1, keepdims=True)
                a_ref[b, t] = (z - mu) / jnp.sqrt(var + 1e-5) * g + be2
                return 0
            lax.fori_loop(0, _T, t_conv, 0)
            return 0

        lax.fori_loop(0, _B, b_body, 0)

    (w1_ref, b1_ref, w2m_ref, b2_ref, w3m_ref, b3_ref) = mlp
    xrows = []
    for b in range(_B):
        m = jnp.mean(a_ref[b, 0], axis=1, keepdims=True)
        xrows.append(_mm00(m, eye))
    xm = jnp.concatenate(xrows, axis=0)
    w1 = w1_ref[...]
    h = _mm11(xm, w1[:, :_N]) + _mm11(imu_ref[...], w1[:, _N:]) + b1_ref[...]
    h = jnp.maximum(h, 0.0)
    h = jnp.maximum(_mm11(h, w2m_ref[...]) + b2_ref[...], 0.0)
    o_ref[...] = _mm11(h, w3m_ref[...]) + b3_ref[...]


def _blocks_call(A0, L5, imu_flat, flat):
    n_in = 3 + len(flat)
    return pl.pallas_call(
        _blocks_body,
        out_shape=jax.ShapeDtypeStruct((_B, 6), _f32),
        scratch_shapes=[
            pltpu.VMEM((_B, _T, _N, _C), _f32),
            pltpu.VMEM((_T, _N, _C), _f32),
        ],
    )(A0, L5, imu_flat, *flat)


def _padF(a, axis):
    pad = [(0, 0)] * a.ndim
    pad[axis] = (0, _C - _F0)
    return jnp.pad(a, pad)


def kernel(x, edge_index, imu_data, params):
    x_tn = jnp.transpose(x, (0, 3, 1, 2)).reshape(_B * _T, _N, _F0)
    pooled, ranks = _pool_call(x_tn, params['pool_w'].reshape(1, _F0))
    A0 = pooled.reshape(_B, _T, _N, _C)
    ranks7 = ranks.reshape(_B, _T, _N)[:, _T - 1, :]
    ei = edge_index.astype(jnp.int32)
    cnt2 = _sc_cnt_call(ei[0], ei[1])
    L5 = _adj_call(cnt2.reshape(2, _N, _N), ranks7)

    flat = []
    first = True
    for bp in params['blocks']:
        if first:
            u2 = _padF(bp['U2'], 0)
            u3 = _padF(bp['U3'].reshape(1, _F0), 1)
            w2 = _padF(bp['W2'], 0)
            w3 = _padF(bp['W3'].reshape(1, _F0), 1)
            th = _padF(bp['theta'], 1)
            wr = _padF(bp['Wr'][:, :, 0, 0], 1)
            first = False
        else:
            u2, w2, th = bp['U2'], bp['W2'], bp['theta']
            u3 = bp['U3'].reshape(1, _C)
            w3 = bp['W3'].reshape(1, _C)
            wr = bp['Wr'][:, :, 0, 0]
        flat += [
            bp['U1'].reshape(1, _N), u2, u3,
            bp['be'][0], bp['Ve'], bp['W1'].reshape(_T, 1), w2, w3,
            bp['bs'][0], bp['Vs'], th, bp['cb'].reshape(1, _C),
            bp['Wt'],
            bp['bt'].reshape(1, _C), wr, bp['br'].reshape(1, _C),
            bp['gamma'].reshape(1, _C), bp['beta'].reshape(1, _C),
        ]
    m = params['mlp']
    flat += [
        m['W1'], m['b1'].reshape(1, 256), m['W2'],
        m['b2'].reshape(1, 128), m['W3'], m['b3'].reshape(1, 6),
    ]
    imu_flat = imu_data.reshape(_B, 6 * _T)
    return _blocks_call(A0, L5, imu_flat, flat)


# speedup vs baseline: 1.8300x; 1.8300x over previous
import jax
import jax.numpy as jnp
from jax import lax
from jax.experimental import pallas as pl
from jax.experimental.pallas import tpu as pltpu
from jax.experimental.pallas import tpu_sc as plsc

_N = 512
_F0 = 8
_T = 8
_B = 4
_E = 8192
_C = 64
_K = 3
_f32 = jnp.float32
_HI = lax.Precision.HIGHEST
_DEF = lax.Precision.DEFAULT

_NBP = 18


def _mm(a, b, prec=_DEF):
    return lax.dot_general(a, b, (((a.ndim - 1,), (0,)), ((), ())),
                           precision=prec, preferred_element_type=_f32)


def _mm00(a, b, prec=_DEF):
    return lax.dot_general(a, b, (((0,), (0,)), ((), ())),
                           precision=prec, preferred_element_type=_f32)


def _mm11(a, b, prec=_DEF):
    return lax.dot_general(a, b, (((1,), (1,)), ((), ())),
                           precision=prec, preferred_element_type=_f32)


def _iota_r(n):
    return lax.broadcasted_iota(jnp.int32, (n, 1), 0).astype(_f32)


def _iota_c(n):
    return lax.broadcasted_iota(jnp.int32, (1, n), 1).astype(_f32)


def _pool_body(x_ref, w_ref, out_ref, rank_ref):
    xb = x_ref[0]
    w = w_ref[...]
    nrm = jnp.sqrt(jnp.sum(w * w))
    s = jnp.tanh(jnp.sum(xb * w, axis=1, keepdims=True) / nrm)
    eye = (_iota_r(_N) == _iota_c(_N)).astype(_f32)
    s_row = _mm00(s, eye)
    gt = (s > s_row).astype(_f32)
    tie = ((s == s_row) & (_iota_r(_N) < _iota_c(_N))).astype(_f32)
    rank = jnp.sum(gt + tie, axis=0, keepdims=True)
    perm = (rank == _iota_r(_N)).astype(_f32)
    pooled = _mm(perm, xb * s)
    out_ref[0] = jnp.concatenate(
        [pooled, jnp.zeros((_N, _C - _F0), _f32)], axis=1)
    rank_ref[0] = rank


def _pool_call(x_tn, w):
    return pl.pallas_call(
        _pool_body,
        grid=(_B * _T,),
        in_specs=[
            pl.BlockSpec((1, _N, _F0), lambda i: (i, 0, 0)),
            pl.BlockSpec((1, _F0), lambda i: (0, 0)),
        ],
        out_specs=[
            pl.BlockSpec((1, _N, _C), lambda i: (i, 0, 0)),
            pl.BlockSpec((1, 1, _N), lambda i: (i, 0, 0)),
        ],
        out_shape=[
            jax.ShapeDtypeStruct((_B * _T, _N, _C), _f32),
            jax.ShapeDtypeStruct((_B * _T, 1, _N), _f32),
        ],
    )(x_tn, w)


_NN = _N * _N
_EPW = _E // 32
_SLC = _NN // 16


def _sc_cnt_body(rows_hbm, cols_hbm, zeros_hbm, out_hbm,
                 rows_v, cols_v, ones_v, cnt_sh):
    c = lax.axis_index("c")
    s = lax.axis_index("s")
    wid = c * 16 + s
    pltpu.sync_copy(zeros_hbm.at[pl.ds(s * _SLC, _SLC)],
                    cnt_sh.at[pl.ds(s * _SLC, _SLC)])
    plsc.subcore_barrier()
    pltpu.sync_copy(rows_hbm.at[pl.ds(wid * _EPW, _EPW)], rows_v)
    pltpu.sync_copy(cols_hbm.at[pl.ds(wid * _EPW, _EPW)], cols_v)
    ones_v[...] = jnp.ones((16,), _f32)

    def g_body(g, carry):
        rvec = rows_v[pl.ds(g * 16, 16)]
        cvec = cols_v[pl.ds(g * 16, 16)]
        idx = rvec * _N + cvec
        pltpu.sync_copy(ones_v, cnt_sh.at[idx], add=True)
        return carry
    lax.fori_loop(0, _EPW // 16, g_body, 0)
    plsc.subcore_barrier()
    pltpu.sync_copy(cnt_sh.at[pl.ds(s * _SLC, _SLC)],
                    out_hbm.at[c, pl.ds(s * _SLC, _SLC)])


def _sc_cnt_call(rows_i, cols_i):
    mesh = plsc.VectorSubcoreMesh(core_axis_name="c", subcore_axis_name="s")
    return pl.kernel(
        _sc_cnt_body,
        out_type=jax.ShapeDtypeStruct((2, _NN), _f32),
        mesh=mesh,
        scratch_types=[
            pltpu.VMEM((_EPW,), jnp.int32),
            pltpu.VMEM((_EPW,), jnp.int32),
            pltpu.VMEM((16,), _f32),
            pltpu.VMEM_SHARED((_NN,), _f32),
        ],
    )(rows_i, cols_i, jnp.zeros((_NN,), _f32))


def _adj_body(c2_ref, rk_ref, l_ref):
    eye = (_iota_r(_N) == _iota_c(_N)).astype(_f32)
    cnt = c2_ref[0] + c2_ref[1]
    deg = jnp.sum(cnt, axis=1, keepdims=True)
    dis = jnp.where(deg > 0, 1.0 / jnp.sqrt(deg), 0.0)
    dis_row = _mm00(dis, eye)
    l_ref[4] = -(dis * dis_row) * cnt
    for t in range(4):
        rk = rk_ref[t:t + 1, :]
        perm = (rk == _iota_r(_N)).astype(_f32)
        tmp = _mm(perm, cnt, _DEF)
        ct = _mm11(tmp, perm, _DEF)
        dt_col = _mm(perm, dis)
        dt_row = _mm00(dt_col, eye)
        l_ref[t] = -(dt_col * dt_row) * ct


def _adj_call(cnt2, ranks7):
    return pl.pallas_call(
        _adj_body,
        grid=(1,),
        in_specs=[
            pl.BlockSpec((2, _N, _N), lambda i: (0, 0, 0)),
            pl.BlockSpec((_B, _N), lambda i: (0, 0)),
        ],
        out_specs=pl.BlockSpec((5, _N, _N), lambda i: (0, 0, 0)),
        out_shape=jax.ShapeDtypeStruct((5, _N, _N), _f32),
    )(cnt2, ranks7)


def _blocks_body(*refs):
    a0_ref, l_ref, imu_ref = refs[0:3]
    bps = refs[3:3 + 3 * _NBP]
    mlp = refs[3 + 3 * _NBP:3 + 3 * _NBP + 7]
    o_ref = refs[3 + 3 * _NBP + 7]
    a_ref, h_ref = refs[3 + 3 * _NBP + 8:]

    eye = (_iota_r(_N) == _iota_c(_N)).astype(_f32)

    for blk in range(3):
        (u1_ref, u2_ref, u3_ref, be_ref, ve_ref, w1_ref, w2_ref, w3_ref,
         bs_ref, vs_ref, th_ref, cb_ref, wt_ref, bt_ref, wr_ref, br_ref,
         g_ref, be2_ref) = bps[_NBP * blk:_NBP * (blk + 1)]
        src = a0_ref if blk == 0 else a_ref
        cb = cb_ref[...]
        btv = bt_ref[...]
        brv = br_ref[...]
        g = g_ref[...]
        be2 = be2_ref[...]
        u1 = u1_ref[...]
        u3 = u3_ref[...]
        w3 = w3_ref[...]

        def b_body(b, _):
            Ab = src[b]
            lhs1 = jnp.sum(Ab * u1[:, :, None], axis=1)
            lhs2 = _mm(lhs1, u2_ref[...])
            rhs = jnp.sum(Ab * u3[None, :, :], axis=2)
            e1 = _mm11(lhs2, rhs)
            esig = jax.nn.sigmoid(e1 + be_ref[...])
            eatt = _mm(ve_ref[...], esig)
            mx = jnp.max(eatt, axis=0, keepdims=True)
            ex = jnp.exp(eatt - mx)
            eatt = ex / jnp.sum(ex, axis=0, keepdims=True)
            c = _mm(eatt, w1_ref[...])
            l2a = jnp.sum(Ab * c[:, :, None], axis=0)
            l2b = _mm(l2a, w2_ref[...])
            r0 = jnp.sum(Ab * w3[None, :, :], axis=2)
            r2 = _mm00(eatt, r0)
            s1 = jax.nn.sigmoid(_mm(l2b, r2) + bs_ref[...])
            s2 = _mm(vs_ref[...], s1)
            mx2 = jnp.max(s2, axis=0, keepdims=True)
            ex2 = jnp.exp(s2 - mx2)
            S = ex2 / jnp.sum(ex2, axis=0, keepdims=True)
            diag = jnp.sum(S * eye, axis=1, keepdims=True)

            def t_cheb(t, _c):
                lt = l_ref[jnp.minimum(t, 4)]
                xt = src[b, t]
                t0 = xt * diag
                out = _mm(t0, th_ref[0]) + cb
                t1 = _mm00(lt * S, t0)
                out = out + _mm(t1, th_ref[1])
                t2 = 2.0 * _mm00(lt, t1) - t0
                out = out + _mm(t2, th_ref[2])
                h_ref[t] = jnp.maximum(out, 0.0)
                return 0
            lax.fori_loop(0, _T, t_cheb, 0)

            def t_conv(t, _c):
                acc = _mm(src[b, t], wr_ref[...])
                for dt in range(3):
                    stp = t + dt - 1
                    valid = jnp.logical_and(stp >= 0, stp < _T)
                    hs = h_ref[jnp.clip(stp, 0, _T - 1)]
                    acc = acc + jnp.where(valid, 1.0, 0.0) * _mm(hs, wt_ref[dt])
                z = jnp.maximum(acc + btv + brv, 0.0)
                mu = jnp.mean(z, axis=1, keepdims=True)
                var = jnp.mean((z - mu) * (z - mu), axis=1, keepdims=True)
                a_ref[b, t] = (z - mu) / jnp.sqrt(var + 1e-5) * g + be2
                return 0
            lax.fori_loop(0, _T, t_conv, 0)
            return 0

        lax.fori_loop(0, _B, b_body, 0)

    (w1a_ref, w1b_ref, b1_ref, w2m_ref, b2_ref, w3m_ref, b3_ref) = mlp
    xrows = []
    for b in range(_B):
        m = jnp.mean(a_ref[b, 0], axis=1, keepdims=True)
        xrows.append(_mm00(m, eye))
    xm = jnp.concatenate(xrows, axis=0)
    h = _mm(xm, w1a_ref[...]) + _mm(imu_ref[...], w1b_ref[...]) + b1_ref[...]
    h = jnp.maximum(h, 0.0)
    h = jnp.maximum(_mm(h, w2m_ref[...]) + b2_ref[...], 0.0)
    o_ref[...] = _mm(h, w3m_ref[...]) + b3_ref[...]


def _blocks_call(A0, L5, imu_flat, flat):
    n_in = 3 + len(flat)
    return pl.pallas_call(
        _blocks_body,
        out_shape=jax.ShapeDtypeStruct((_B, 6), _f32),
        scratch_shapes=[
            pltpu.VMEM((_B, _T, _N, _C), _f32),
            pltpu.VMEM((_T, _N, _C), _f32),
        ],
    )(A0, L5, imu_flat, *flat)


def _padF(a, axis):
    pad = [(0, 0)] * a.ndim
    pad[axis] = (0, _C - _F0)
    return jnp.pad(a, pad)


def kernel(x, edge_index, imu_data, params):
    x_tn = jnp.transpose(x, (0, 3, 1, 2)).reshape(_B * _T, _N, _F0)
    pooled, ranks = _pool_call(x_tn, params['pool_w'].reshape(1, _F0))
    A0 = pooled.reshape(_B, _T, _N, _C)
    ranks7 = ranks.reshape(_B, _T, _N)[:, _T - 1, :]
    ei = edge_index.astype(jnp.int32)
    cnt2 = _sc_cnt_call(ei[0], ei[1])
    L5 = _adj_call(cnt2.reshape(2, _N, _N), ranks7)

    flat = []
    first = True
    for bp in params['blocks']:
        if first:
            u2 = _padF(bp['U2'], 0)
            u3 = _padF(bp['U3'].reshape(1, _F0), 1)
            w2 = _padF(bp['W2'], 0)
            w3 = _padF(bp['W3'].reshape(1, _F0), 1)
            th = _padF(bp['theta'], 1)
            wr = _padF(jnp.transpose(bp['Wr'][:, :, 0, 0]), 0)
            first = False
        else:
            u2, w2, th = bp['U2'], bp['W2'], bp['theta']
            u3 = bp['U3'].reshape(1, _C)
            w3 = bp['W3'].reshape(1, _C)
            wr = jnp.transpose(bp['Wr'][:, :, 0, 0])
        flat += [
            bp['U1'].reshape(1, _N), u2, u3,
            bp['be'][0], bp['Ve'], bp['W1'].reshape(_T, 1), w2, w3,
            bp['bs'][0], bp['Vs'], th, bp['cb'].reshape(1, _C),
            jnp.transpose(bp['Wt'][:, :, 0, :], (2, 1, 0)),
            bp['bt'].reshape(1, _C), wr, bp['br'].reshape(1, _C),
            bp['gamma'].reshape(1, _C), bp['beta'].reshape(1, _C),
        ]
    m = params['mlp']
    flat += [
        jnp.transpose(m['W1'][:, :_N]), jnp.transpose(m['W1'][:, _N:]),
        m['b1'].reshape(1, 256), jnp.transpose(m['W2']),
        m['b2'].reshape(1, 128), jnp.transpose(m['W3']),
        m['b3'].reshape(1, 6),
    ]
    imu_flat = imu_data.reshape(_B, 6 * _T)
    return _blocks_call(A0, L5, imu_flat, flat)
